# baseline (speedup 1.0000x reference)
import functools

import jax
import jax.numpy as jnp
from jax import lax
from jax.experimental import pallas as pl
from jax.experimental.pallas import tpu as pltpu
from jax.experimental.pallas import tpu_sc as plsc

_NC = 2
_NS = 16
_NW = _NC * _NS


def _precompute_tables(emb, wt, d_pad):
    n, d = emb.shape
    blk = 2000
    assert n % blk == 0
    half = d_pad // 2

    def pack(q):
        u = jax.lax.bitcast_convert_type(q, jnp.uint32)
        rnd = (u + jnp.uint32(0x7FFF) + ((u >> 16) & jnp.uint32(1))) >> 16
        word = rnd[:, :half] | (rnd[:, half:] << 16)
        return jax.lax.bitcast_convert_type(word, jnp.int32)

    def body(x_ref, wt_ref, u_ref, v_ref):
        x = jnp.clip(x_ref[...], -10.0, 10.0).astype(jnp.bfloat16)
        p = jnp.dot(x, wt_ref[...].astype(jnp.bfloat16),
                    preferred_element_type=jnp.float32)
        u_ref[...] = pack(p[:, :d_pad])
        v_ref[...] = pack(p[:, d_pad:])

    return pl.pallas_call(
        body,
        grid=(n // blk,),
        in_specs=[
            pl.BlockSpec((blk, d), lambda i: (i, 0)),
            pl.BlockSpec((d, 2 * d_pad), lambda i: (0, 0)),
        ],
        out_specs=[
            pl.BlockSpec((blk, half), lambda i: (i, 0)),
            pl.BlockSpec((blk, half), lambda i: (i, 0)),
        ],
        out_shape=[
            jax.ShapeDtypeStruct((n, half), jnp.int32),
            jax.ShapeDtypeStruct((n, half), jnp.int32),
        ],
    )(emb, wt)


def _sc_gather(u, v, src, tgt):
    e_total = src.shape[0]
    d = u.shape[1]
    b_per_w = e_total // _NW
    chunk = 64
    n_full = b_per_w // chunk
    tail = b_per_w - n_full * chunk
    assert tail % 8 == 0 and e_total % _NW == 0 and n_full % 2 == 0

    mesh = plsc.VectorSubcoreMesh(
        core_axis_name="c", subcore_axis_name="s",
        num_cores=_NC, num_subcores=_NS)

    @functools.partial(
        pl.kernel,
        out_type=(
            jax.ShapeDtypeStruct((e_total, d), jnp.int32),
            jax.ShapeDtypeStruct((e_total, d), jnp.int32),
        ),
        mesh=mesh,
        scratch_types=[
            pltpu.VMEM((b_per_w,), jnp.int32),
            pltpu.VMEM((b_per_w,), jnp.int32),
            pltpu.VMEM((2, chunk, d), jnp.int32),
            pltpu.VMEM((2, chunk, d), jnp.int32),
            pltpu.SemaphoreType.DMA((2,)),
            pltpu.SemaphoreType.DMA((2,)),
            pltpu.SemaphoreType.DMA((2,)),
            pltpu.SemaphoreType.DMA((2,)),
        ],
    )
    def k(u_hbm, v_hbm, src_hbm, tgt_hbm, gu_hbm, gv_hbm,
          idx_s, idx_t, rows_u, rows_v, gsem_u, gsem_v, wsem_u, wsem_v):
        wid = lax.axis_index("s") * _NC + lax.axis_index("c")
        base0 = wid * b_per_w
        pltpu.sync_copy(src_hbm.at[pl.ds(base0, b_per_w)], idx_s)
        pltpu.sync_copy(tgt_hbm.at[pl.ds(base0, b_per_w)], idx_t)

        def start_gather(ci, p):
            off = ci * chunk
            pltpu.async_copy(
                u_hbm.at[idx_s.at[pl.ds(off, chunk)]], rows_u.at[p],
                gsem_u.at[p])
            pltpu.async_copy(
                v_hbm.at[idx_t.at[pl.ds(off, chunk)]], rows_v.at[p],
                gsem_v.at[p])

        def wait_gather(p):
            pltpu.make_async_copy(
                u_hbm.at[idx_s.at[pl.ds(0, chunk)]], rows_u.at[p],
                gsem_u.at[p]).wait()
            pltpu.make_async_copy(
                v_hbm.at[idx_t.at[pl.ds(0, chunk)]], rows_v.at[p],
                gsem_v.at[p]).wait()

        def start_write(ci, p):
            base = base0 + ci * chunk
            pltpu.async_copy(rows_u.at[p], gu_hbm.at[pl.ds(base, chunk)],
                             wsem_u.at[p])
            pltpu.async_copy(rows_v.at[p], gv_hbm.at[pl.ds(base, chunk)],
                             wsem_v.at[p])

        def wait_write(p):
            pltpu.make_async_copy(rows_u.at[p], gu_hbm.at[pl.ds(base0, chunk)],
                                  wsem_u.at[p]).wait()
            pltpu.make_async_copy(rows_v.at[p], gv_hbm.at[pl.ds(base0, chunk)],
                                  wsem_v.at[p]).wait()

        start_gather(0, 0)

        def loop_body(i, carry):
            p = lax.rem(i, 2)
            q = 1 - p

            @pl.when(i + 1 < n_full)
            def _():
                @pl.when(i >= 1)
                def _():
                    wait_write(q)
                start_gather(i + 1, q)

            wait_gather(p)
            start_write(i, p)
            return carry

        lax.fori_loop(0, n_full, loop_body, 0)
        wait_write(0)
        wait_write(1)

        if tail:
            base = base0 + n_full * chunk
            ru = rows_u.at[0].at[pl.ds(0, tail)]
            rv = rows_v.at[0].at[pl.ds(0, tail)]
            pltpu.async_copy(
                u_hbm.at[idx_s.at[pl.ds(n_full * chunk, tail)]], ru,
                gsem_u.at[0]).wait()
            pltpu.async_copy(
                v_hbm.at[idx_t.at[pl.ds(n_full * chunk, tail)]], rv,
                gsem_v.at[0]).wait()
            pltpu.sync_copy(ru, gu_hbm.at[pl.ds(base, tail)])
            pltpu.sync_copy(rv, gv_hbm.at[pl.ds(base, tail)])

    return k(u, v, src, tgt)


def _edge_mlp(gu, gv, e_b1, e_g1, e_bt1, w2t, e_b2, e_g2, e_bt2, w3, e_b3,
              a_b1, a_g1, a_bt1, aw2t, a_b2, a_g2, a_bt2, aw3t, a_b3,
              d_edge, d_army):
    e_total, d = gu.shape
    blk = 640
    assert e_total % blk == 0

    def ln(x, gam, bet):
        m = jnp.mean(x, axis=-1, keepdims=True)
        v = jnp.mean((x - m) ** 2, axis=-1, keepdims=True)
        return (x - m) * lax.rsqrt(v + 1e-5) * gam + bet

    def unpack(w_i32):
        w = jax.lax.bitcast_convert_type(w_i32, jnp.uint32)
        xe = jax.lax.bitcast_convert_type(w << 16, jnp.float32)
        xa = jax.lax.bitcast_convert_type(
            w[:, :d_army] & jnp.uint32(0xFFFF0000), jnp.float32)
        return xe, xa

    def body(gu_ref, gv_ref, eb1, eg1, ebt1, w2t_ref, eb2, eg2, ebt2,
             w3_ref, eb3, ab1, ag1, abt1, aw2t_ref, ab2, ag2, abt2,
             aw3t_ref, ab3, edge_ref, army_ref):
        xe_u, xa_u = unpack(gu_ref[...])
        xe_v, xa_v = unpack(gv_ref[...])
        xe = xe_u + xe_v + eb1[...]
        xa = xa_u + xa_v + ab1[...]

        he = jax.nn.relu(ln(xe, eg1[...], ebt1[...]))
        he = jnp.dot(he.astype(jnp.bfloat16), w2t_ref[...].astype(jnp.bfloat16),
                     preferred_element_type=jnp.float32) + eb2[...]
        he = jax.nn.relu(ln(he, eg2[...], ebt2[...]))
        edge = jnp.sum(he * w3_ref[...], axis=-1) + eb3[0]
        edge_ref[0, 0, :] = jnp.clip(edge, -20.0, 20.0)

        ha = jax.nn.relu(ln(xa, ag1[...], abt1[...]))
        ha = jnp.dot(ha.astype(jnp.bfloat16), aw2t_ref[...].astype(jnp.bfloat16),
                     preferred_element_type=jnp.float32) + ab2[...]
        ha = jax.nn.relu(ln(ha, ag2[...], abt2[...]))
        army = jnp.dot(ha.astype(jnp.bfloat16), aw3t_ref[...].astype(jnp.bfloat16),
                       preferred_element_type=jnp.float32) + ab3[...]
        army_ref[...] = jnp.clip(army, -20.0, 20.0)

    def full(x):
        return pl.BlockSpec(x.shape, lambda i: (0,) * x.ndim)

    weights = [e_b1, e_g1, e_bt1, w2t, e_b2, e_g2, e_bt2, w3, e_b3,
               a_b1, a_g1, a_bt1, aw2t, a_b2, a_g2, a_bt2, aw3t, a_b3]

    return pl.pallas_call(
        body,
        grid=(e_total // blk,),
        in_specs=[
            pl.BlockSpec((blk, d), lambda i: (i, 0)),
            pl.BlockSpec((blk, d), lambda i: (i, 0)),
        ] + [full(w) for w in weights],
        out_specs=[
            pl.BlockSpec((1, 1, blk), lambda i: (i, 0, 0)),
            pl.BlockSpec((blk, 4), lambda i: (i, 0)),
        ],
        out_shape=[
            jax.ShapeDtypeStruct((e_total // blk, 1, blk), jnp.float32),
            jax.ShapeDtypeStruct((e_total, 4), jnp.float32),
        ],
    )(gu, gv, *weights)


def kernel(node_embeddings, action_edges,
           e_W1, e_b1, e_g1, e_bt1, e_W2, e_b2, e_g2, e_bt2, e_W3, e_b3,
           a_W1, a_b1, a_g1, a_bt1, a_W2, a_b2, a_g2, a_bt2, a_W3, a_b3):
    d = node_embeddings.shape[1]
    d_edge = e_W1.shape[0]
    d_army = a_W1.shape[0]
    d_pad = 2 * d_edge

    zpad = jnp.zeros((d, d_pad - d_edge - d_army), jnp.float32)
    wt = jnp.concatenate(
        [e_W1[:, :d].T, a_W1[:, :d].T, zpad,
         e_W1[:, d:].T, a_W1[:, d:].T, zpad], axis=1)

    u, v = _precompute_tables(node_embeddings, wt, d_pad)

    src = action_edges[:, 0]
    tgt = action_edges[:, 1]
    gu, gv = _sc_gather(u, v, src, tgt)

    edge_logits, army_logits = _edge_mlp(
        gu, gv, e_b1, e_g1, e_bt1, e_W2.T, e_b2, e_g2, e_bt2, e_W3[0], e_b3,
        a_b1, a_g1, a_bt1, a_W2.T, a_b2, a_g2, a_bt2, a_W3.T, a_b3,
        d_edge, d_army)
    return (edge_logits.reshape(-1), army_logits)


# speedup vs baseline: 1.0955x; 1.0955x over previous
import functools

import jax
import jax.numpy as jnp
from jax import lax
from jax.experimental import pallas as pl
from jax.experimental.pallas import tpu as pltpu
from jax.experimental.pallas import tpu_sc as plsc

_NC = 2
_NS = 16
_NW = _NC * _NS


def _precompute_tables(emb, wt, d_pad):
    n, d = emb.shape
    blk = 2000
    assert n % blk == 0
    half = d_pad // 2

    def pack(q):
        u = jax.lax.bitcast_convert_type(q, jnp.uint32)
        rnd = (u + jnp.uint32(0x7FFF) + ((u >> 16) & jnp.uint32(1))) >> 16
        word = (rnd[:, :half] << 16) | rnd[:, half:]
        return jax.lax.bitcast_convert_type(word, jnp.int32)

    def body(x_ref, wt_ref, u_ref, v_ref):
        x = jnp.clip(x_ref[...], -10.0, 10.0).astype(jnp.bfloat16)
        p = jnp.dot(x, wt_ref[...].astype(jnp.bfloat16),
                    preferred_element_type=jnp.float32)
        u_ref[...] = pack(p[:, :d_pad])
        v_ref[...] = pack(p[:, d_pad:])

    return pl.pallas_call(
        body,
        grid=(n // blk,),
        in_specs=[
            pl.BlockSpec((blk, d), lambda i: (i, 0)),
            pl.BlockSpec((d, 2 * d_pad), lambda i: (0, 0)),
        ],
        out_specs=[
            pl.BlockSpec((blk, half), lambda i: (i, 0)),
            pl.BlockSpec((blk, half), lambda i: (i, 0)),
        ],
        out_shape=[
            jax.ShapeDtypeStruct((n, half), jnp.int32),
            jax.ShapeDtypeStruct((n, half), jnp.int32),
        ],
    )(emb, wt)


def _sc_gather(u, v, src, tgt):
    e_total = src.shape[0]
    d = u.shape[1]
    b_per_w = e_total // _NW
    chunk = 64
    n_full = b_per_w // chunk
    tail = b_per_w - n_full * chunk
    assert tail % 8 == 0 and e_total % _NW == 0 and n_full % 2 == 0

    mesh = plsc.VectorSubcoreMesh(
        core_axis_name="c", subcore_axis_name="s",
        num_cores=_NC, num_subcores=_NS)

    @functools.partial(
        pl.kernel,
        out_type=(
            jax.ShapeDtypeStruct((e_total, d), jnp.int32),
            jax.ShapeDtypeStruct((e_total, d), jnp.int32),
        ),
        mesh=mesh,
        scratch_types=[
            pltpu.VMEM((b_per_w,), jnp.int32),
            pltpu.VMEM((b_per_w,), jnp.int32),
            pltpu.VMEM((2, chunk, d), jnp.int32),
            pltpu.VMEM((2, chunk, d), jnp.int32),
            pltpu.SemaphoreType.DMA((2,)),
            pltpu.SemaphoreType.DMA((2,)),
            pltpu.SemaphoreType.DMA((2,)),
            pltpu.SemaphoreType.DMA((2,)),
        ],
    )
    def k(u_hbm, v_hbm, src_hbm, tgt_hbm, gu_hbm, gv_hbm,
          idx_s, idx_t, rows_u, rows_v, gsem_u, gsem_v, wsem_u, wsem_v):
        wid = lax.axis_index("s") * _NC + lax.axis_index("c")
        base0 = wid * b_per_w
        pltpu.sync_copy(src_hbm.at[pl.ds(base0, b_per_w)], idx_s)
        pltpu.sync_copy(tgt_hbm.at[pl.ds(base0, b_per_w)], idx_t)

        def start_gather(ci, p):
            off = ci * chunk
            pltpu.async_copy(
                u_hbm.at[idx_s.at[pl.ds(off, chunk)]], rows_u.at[p],
                gsem_u.at[p])
            pltpu.async_copy(
                v_hbm.at[idx_t.at[pl.ds(off, chunk)]], rows_v.at[p],
                gsem_v.at[p])

        def wait_gather(p):
            pltpu.make_async_copy(
                u_hbm.at[idx_s.at[pl.ds(0, chunk)]], rows_u.at[p],
                gsem_u.at[p]).wait()
            pltpu.make_async_copy(
                v_hbm.at[idx_t.at[pl.ds(0, chunk)]], rows_v.at[p],
                gsem_v.at[p]).wait()

        def start_write(ci, p):
            base = base0 + ci * chunk
            pltpu.async_copy(rows_u.at[p], gu_hbm.at[pl.ds(base, chunk)],
                             wsem_u.at[p])
            pltpu.async_copy(rows_v.at[p], gv_hbm.at[pl.ds(base, chunk)],
                             wsem_v.at[p])

        def wait_write(p):
            pltpu.make_async_copy(rows_u.at[p], gu_hbm.at[pl.ds(base0, chunk)],
                                  wsem_u.at[p]).wait()
            pltpu.make_async_copy(rows_v.at[p], gv_hbm.at[pl.ds(base0, chunk)],
                                  wsem_v.at[p]).wait()

        start_gather(0, 0)

        def loop_body(i, carry):
            p = lax.rem(i, 2)
            q = 1 - p

            @pl.when(i + 1 < n_full)
            def _():
                @pl.when(i >= 1)
                def _():
                    wait_write(q)
                start_gather(i + 1, q)

            wait_gather(p)
            start_write(i, p)
            return carry

        lax.fori_loop(0, n_full, loop_body, 0)
        wait_write(0)
        wait_write(1)

        if tail:
            base = base0 + n_full * chunk
            ru = rows_u.at[0].at[pl.ds(0, tail)]
            rv = rows_v.at[0].at[pl.ds(0, tail)]
            pltpu.async_copy(
                u_hbm.at[idx_s.at[pl.ds(n_full * chunk, tail)]], ru,
                gsem_u.at[0]).wait()
            pltpu.async_copy(
                v_hbm.at[idx_t.at[pl.ds(n_full * chunk, tail)]], rv,
                gsem_v.at[0]).wait()
            pltpu.sync_copy(ru, gu_hbm.at[pl.ds(base, tail)])
            pltpu.sync_copy(rv, gv_hbm.at[pl.ds(base, tail)])

    return k(u, v, src, tgt)


def _edge_mlp(gu, gv, e_b1, e_g1, e_bt1, w2t, e_b2, e_g2, e_bt2, w3, e_b3,
              a_b1, a_g1, a_bt1, aw2t, a_b2, a_g2, a_bt2, aw3t, a_b3,
              d_edge, d_army):
    e_total, d = gu.shape
    blk = 640
    assert e_total % blk == 0

    def ln(x, gam, bet):
        m = jnp.mean(x, axis=-1, keepdims=True)
        v = jnp.mean((x - m) ** 2, axis=-1, keepdims=True)
        return (x - m) * (lax.rsqrt(v + 1e-5) * gam) + bet

    def unpack(w_i32):
        xe = jax.lax.bitcast_convert_type(w_i32, jnp.float32)
        xa = jax.lax.bitcast_convert_type(
            jax.lax.bitcast_convert_type(w_i32[:, :d_army], jnp.uint32) << 16,
            jnp.float32)
        return xe, xa

    def body(gu_ref, gv_ref, eb1, eg1, ebt1, w2t_ref, eb2, eg2, ebt2,
             w3_ref, eb3, ab1, ag1, abt1, aw2t_ref, ab2, ag2, abt2,
             aw3t_ref, ab3, edge_ref, army_ref):
        xe_u, xa_u = unpack(gu_ref[...])
        xe_v, xa_v = unpack(gv_ref[...])
        xe = xe_u + xe_v + eb1[...]
        xa = xa_u + xa_v + ab1[...]

        he = jax.nn.relu(ln(xe, eg1[...], ebt1[...]))
        he = jnp.dot(he.astype(jnp.bfloat16), w2t_ref[...].astype(jnp.bfloat16),
                     preferred_element_type=jnp.float32) + eb2[...]
        he = jax.nn.relu(ln(he, eg2[...], ebt2[...]))
        edge = jnp.sum(he * w3_ref[...], axis=-1) + eb3[0]
        edge_ref[0, 0, :] = jnp.clip(edge, -20.0, 20.0)

        ha = jax.nn.relu(ln(xa, ag1[...], abt1[...]))
        ha = jnp.dot(ha.astype(jnp.bfloat16), aw2t_ref[...].astype(jnp.bfloat16),
                     preferred_element_type=jnp.float32) + ab2[...]
        ha = jax.nn.relu(ln(ha, ag2[...], abt2[...]))
        army = jnp.dot(ha.astype(jnp.bfloat16), aw3t_ref[...].astype(jnp.bfloat16),
                       preferred_element_type=jnp.float32) + ab3[...]
        army_ref[0, :, :] = jnp.clip(army, -20.0, 20.0).T

    def full(x):
        return pl.BlockSpec(x.shape, lambda i: (0,) * x.ndim)

    weights = [e_b1, e_g1, e_bt1, w2t, e_b2, e_g2, e_bt2, w3, e_b3,
               a_b1, a_g1, a_bt1, aw2t, a_b2, a_g2, a_bt2, aw3t, a_b3]

    return pl.pallas_call(
        body,
        grid=(e_total // blk,),
        in_specs=[
            pl.BlockSpec((blk, d), lambda i: (i, 0)),
            pl.BlockSpec((blk, d), lambda i: (i, 0)),
        ] + [full(w) for w in weights],
        out_specs=[
            pl.BlockSpec((1, 1, blk), lambda i: (i, 0, 0)),
            pl.BlockSpec((1, 4, blk), lambda i: (i, 0, 0)),
        ],
        out_shape=[
            jax.ShapeDtypeStruct((e_total // blk, 1, blk), jnp.float32),
            jax.ShapeDtypeStruct((e_total // blk, 4, blk), jnp.float32),
        ],
    )(gu, gv, *weights)


def kernel(node_embeddings, action_edges,
           e_W1, e_b1, e_g1, e_bt1, e_W2, e_b2, e_g2, e_bt2, e_W3, e_b3,
           a_W1, a_b1, a_g1, a_bt1, a_W2, a_b2, a_g2, a_bt2, a_W3, a_b3):
    d = node_embeddings.shape[1]
    d_edge = e_W1.shape[0]
    d_army = a_W1.shape[0]
    d_pad = 2 * d_edge

    zpad = jnp.zeros((d, d_pad - d_edge - d_army), jnp.float32)
    wt = jnp.concatenate(
        [e_W1[:, :d].T, a_W1[:, :d].T, zpad,
         e_W1[:, d:].T, a_W1[:, d:].T, zpad], axis=1)

    u, v = _precompute_tables(node_embeddings, wt, d_pad)

    src = action_edges[:, 0]
    tgt = action_edges[:, 1]
    gu, gv = _sc_gather(u, v, src, tgt)

    edge_logits, army_t = _edge_mlp(
        gu, gv, e_b1, e_g1, e_bt1, e_W2.T, e_b2, e_g2, e_bt2, e_W3[0], e_b3,
        a_b1, a_g1, a_bt1, a_W2.T, a_b2, a_g2, a_bt2, a_W3.T, a_b3,
        d_edge, d_army)
    e_total = action_edges.shape[0]
    army_logits = army_t.transpose(0, 2, 1).reshape(e_total, 4)
    return (edge_logits.reshape(-1), army_logits)


# speedup vs baseline: 1.2030x; 1.0981x over previous
import functools

import jax
import jax.numpy as jnp
from jax import lax
from jax.experimental import pallas as pl
from jax.experimental.pallas import tpu as pltpu
from jax.experimental.pallas import tpu_sc as plsc

_NC = 2
_NS = 16
_NW = _NC * _NS


def _precompute_tables(emb, wt, d_pad):
    n, d = emb.shape
    blk = 2000
    assert n % blk == 0
    half = d_pad // 2

    def pack(q):
        u = jax.lax.bitcast_convert_type(q, jnp.uint32)
        rnd = (u + jnp.uint32(0x7FFF) + ((u >> 16) & jnp.uint32(1))) >> 16
        word = (rnd[:, :half] << 16) | rnd[:, half:]
        return jax.lax.bitcast_convert_type(word, jnp.int32)

    def body(x_ref, wt_ref, u_ref, v_ref):
        x = jnp.clip(x_ref[...], -10.0, 10.0).astype(jnp.bfloat16)
        p = jnp.dot(x, wt_ref[...].astype(jnp.bfloat16),
                    preferred_element_type=jnp.float32)
        u_ref[...] = pack(p[:, :d_pad])
        v_ref[...] = pack(p[:, d_pad:])

    return pl.pallas_call(
        body,
        grid=(n // blk,),
        in_specs=[
            pl.BlockSpec((blk, d), lambda i: (i, 0)),
            pl.BlockSpec((d, 2 * d_pad), lambda i: (0, 0)),
        ],
        out_specs=[
            pl.BlockSpec((blk, half), lambda i: (i, 0)),
            pl.BlockSpec((blk, half), lambda i: (i, 0)),
        ],
        out_shape=[
            jax.ShapeDtypeStruct((n, half), jnp.int32),
            jax.ShapeDtypeStruct((n, half), jnp.int32),
        ],
    )(emb, wt)


def _sc_gather(u, v, src, tgt):
    e_total = src.shape[0]
    d = u.shape[1]
    b_per_w = e_total // _NW
    chunk = 64
    n_full = b_per_w // chunk
    tail = b_per_w - n_full * chunk
    assert tail % 8 == 0 and e_total % _NW == 0 and n_full >= 2

    mesh = plsc.VectorSubcoreMesh(
        core_axis_name="c", subcore_axis_name="s",
        num_cores=_NC, num_subcores=_NS)

    @functools.partial(
        pl.kernel,
        out_type=(
            jax.ShapeDtypeStruct((e_total, d), jnp.int32),
            jax.ShapeDtypeStruct((e_total, d), jnp.int32),
        ),
        mesh=mesh,
        scratch_types=[
            pltpu.VMEM((b_per_w,), jnp.int32),
            pltpu.VMEM((b_per_w,), jnp.int32),
            pltpu.VMEM((2, chunk, d), jnp.int32),
            pltpu.VMEM((2, chunk, d), jnp.int32),
            pltpu.SemaphoreType.DMA((2,)),
            pltpu.SemaphoreType.DMA((2,)),
            pltpu.SemaphoreType.DMA((2,)),
            pltpu.SemaphoreType.DMA((2,)),
        ],
    )
    def k(u_hbm, v_hbm, src_hbm, tgt_hbm, gu_hbm, gv_hbm,
          idx_s, idx_t, rows_u, rows_v, gsem_u, gsem_v, wsem_u, wsem_v):
        wid = lax.axis_index("s") * _NC + lax.axis_index("c")
        base0 = wid * b_per_w
        pltpu.sync_copy(src_hbm.at[pl.ds(base0, b_per_w)], idx_s)
        pltpu.sync_copy(tgt_hbm.at[pl.ds(base0, b_per_w)], idx_t)

        def start_gather(ci, p):
            off = ci * chunk
            pltpu.async_copy(
                u_hbm.at[idx_s.at[pl.ds(off, chunk)]], rows_u.at[p],
                gsem_u.at[p])
            pltpu.async_copy(
                v_hbm.at[idx_t.at[pl.ds(off, chunk)]], rows_v.at[p],
                gsem_v.at[p])

        def wait_gather(p):
            pltpu.make_async_copy(
                u_hbm.at[idx_s.at[pl.ds(0, chunk)]], rows_u.at[p],
                gsem_u.at[p]).wait()
            pltpu.make_async_copy(
                v_hbm.at[idx_t.at[pl.ds(0, chunk)]], rows_v.at[p],
                gsem_v.at[p]).wait()

        def start_write(ci, p):
            base = base0 + ci * chunk
            pltpu.async_copy(rows_u.at[p], gu_hbm.at[pl.ds(base, chunk)],
                             wsem_u.at[p])
            pltpu.async_copy(rows_v.at[p], gv_hbm.at[pl.ds(base, chunk)],
                             wsem_v.at[p])

        def wait_write(p):
            pltpu.make_async_copy(rows_u.at[p], gu_hbm.at[pl.ds(base0, chunk)],
                                  wsem_u.at[p]).wait()
            pltpu.make_async_copy(rows_v.at[p], gv_hbm.at[pl.ds(base0, chunk)],
                                  wsem_v.at[p]).wait()

        start_gather(0, 0)

        def loop_body(i, carry):
            p = lax.rem(i, 2)
            q = 1 - p

            @pl.when(i + 1 < n_full)
            def _():
                @pl.when(i >= 1)
                def _():
                    wait_write(q)
                start_gather(i + 1, q)

            wait_gather(p)
            start_write(i, p)
            return carry

        lax.fori_loop(0, n_full, loop_body, 0)
        wait_write(0)
        wait_write(1)

        if tail:
            base = base0 + n_full * chunk
            ru = rows_u.at[0].at[pl.ds(0, tail)]
            rv = rows_v.at[0].at[pl.ds(0, tail)]
            pltpu.async_copy(
                u_hbm.at[idx_s.at[pl.ds(n_full * chunk, tail)]], ru,
                gsem_u.at[0]).wait()
            pltpu.async_copy(
                v_hbm.at[idx_t.at[pl.ds(n_full * chunk, tail)]], rv,
                gsem_v.at[0]).wait()
            pltpu.sync_copy(ru, gu_hbm.at[pl.ds(base, tail)])
            pltpu.sync_copy(rv, gv_hbm.at[pl.ds(base, tail)])

    return k(u, v, src, tgt)


def _edge_mlp(gu, gv, e_b1, e_g1, e_bt1, w2t, e_b2, e_g2, e_bt2, w3, e_b3,
              a_b1, a_g1, a_bt1, aw2t, a_b2, a_g2, a_bt2, aw3t, a_b3,
              d_edge, d_army):
    e_total, d = gu.shape
    blk = 640
    assert e_total % blk == 0

    def ln(x, gam, bet):
        m = jnp.mean(x, axis=-1, keepdims=True)
        v = jnp.mean((x - m) ** 2, axis=-1, keepdims=True)
        return (x - m) * (lax.rsqrt(v + 1e-5) * gam) + bet

    def unpack(w_i32):
        xe = jax.lax.bitcast_convert_type(w_i32, jnp.float32)
        xa = jax.lax.bitcast_convert_type(
            jax.lax.bitcast_convert_type(w_i32[:, :d_army], jnp.uint32) << 16,
            jnp.float32)
        return xe, xa

    def body(gu_ref, gv_ref, eb1, eg1, ebt1, w2t_ref, eb2, eg2, ebt2,
             w3_ref, eb3, ab1, ag1, abt1, aw2t_ref, ab2, ag2, abt2,
             aw3t_ref, ab3, edge_ref, army_ref):
        xe_u, xa_u = unpack(gu_ref[...])
        xe_v, xa_v = unpack(gv_ref[...])
        xe = xe_u + xe_v + eb1[...]
        xa = xa_u + xa_v + ab1[...]

        he = jax.nn.relu(ln(xe, eg1[...], ebt1[...]))
        he = jnp.dot(he.astype(jnp.bfloat16), w2t_ref[...].astype(jnp.bfloat16),
                     preferred_element_type=jnp.float32) + eb2[...]
        he = jax.nn.relu(ln(he, eg2[...], ebt2[...]))
        edge = jnp.sum(he * w3_ref[...], axis=-1) + eb3[0]
        edge_ref[0, 0, :] = jnp.clip(edge, -20.0, 20.0)

        ha = jax.nn.relu(ln(xa, ag1[...], abt1[...]))
        ha = jnp.dot(ha.astype(jnp.bfloat16), aw2t_ref[...].astype(jnp.bfloat16),
                     preferred_element_type=jnp.float32) + ab2[...]
        ha = jax.nn.relu(ln(ha, ag2[...], abt2[...]))
        army = jnp.dot(ha.astype(jnp.bfloat16), aw3t_ref[...].astype(jnp.bfloat16),
                       preferred_element_type=jnp.float32) + ab3[...]
        army_ref[0, :, :] = jnp.clip(army, -20.0, 20.0).T

    def full(x):
        return pl.BlockSpec(x.shape, lambda i: (0,) * x.ndim)

    weights = [e_b1, e_g1, e_bt1, w2t, e_b2, e_g2, e_bt2, w3, e_b3,
               a_b1, a_g1, a_bt1, aw2t, a_b2, a_g2, a_bt2, aw3t, a_b3]

    return pl.pallas_call(
        body,
        grid=(e_total // blk,),
        in_specs=[
            pl.BlockSpec((blk, d), lambda i: (i, 0)),
            pl.BlockSpec((blk, d), lambda i: (i, 0)),
        ] + [full(w) for w in weights],
        out_specs=[
            pl.BlockSpec((1, 1, blk), lambda i: (i, 0, 0)),
            pl.BlockSpec((1, 4, blk), lambda i: (i, 0, 0)),
        ],
        out_shape=[
            jax.ShapeDtypeStruct((e_total // blk, 1, blk), jnp.float32),
            jax.ShapeDtypeStruct((e_total // blk, 4, blk), jnp.float32),
        ],
    )(gu, gv, *weights)


def kernel(node_embeddings, action_edges,
           e_W1, e_b1, e_g1, e_bt1, e_W2, e_b2, e_g2, e_bt2, e_W3, e_b3,
           a_W1, a_b1, a_g1, a_bt1, a_W2, a_b2, a_g2, a_bt2, a_W3, a_b3):
    d = node_embeddings.shape[1]
    d_edge = e_W1.shape[0]
    d_army = a_W1.shape[0]
    d_pad = 2 * d_edge

    zpad = jnp.zeros((d, d_pad - d_edge - d_army), jnp.float32)
    wt = jnp.concatenate(
        [e_W1[:, :d].T, a_W1[:, :d].T, zpad,
         e_W1[:, d:].T, a_W1[:, d:].T, zpad], axis=1)

    u, v = _precompute_tables(node_embeddings, wt, d_pad)

    src = action_edges[:, 0]
    tgt = action_edges[:, 1]

    e_total = action_edges.shape[0]
    quantum = 10240
    seg0 = ((e_total // 2) // quantum + 1) * quantum
    seg_sizes = [seg0, e_total - seg0]

    edges, armies = [], []
    off = 0
    for es in seg_sizes:
        gu, gv = _sc_gather(u, v, src[off:off + es], tgt[off:off + es])
        e_log, a_t = _edge_mlp(
            gu, gv, e_b1, e_g1, e_bt1, e_W2.T, e_b2, e_g2, e_bt2, e_W3[0],
            e_b3, a_b1, a_g1, a_bt1, a_W2.T, a_b2, a_g2, a_bt2, a_W3.T, a_b3,
            d_edge, d_army)
        edges.append(e_log.reshape(-1))
        armies.append(a_t.transpose(0, 2, 1).reshape(es, 4))
        off += es

    return (jnp.concatenate(edges), jnp.concatenate(armies, axis=0))
